# baseline (speedup 1.0000x reference)
import functools

import jax
import jax.numpy as jnp
from jax import lax
from jax.experimental import pallas as pl
from jax.experimental.pallas import tpu as pltpu
from jax.experimental.pallas import tpu_sc as plsc

NC = 2
NS = 16
NW = NC * NS
CHUNK = 128
LW = 128


def _sc_mesh():
    return plsc.VectorSubcoreMesh(core_axis_name="c", subcore_axis_name="s")


def _degree_sc(dst_p, ones_r, zeros_r, n_pad, q0, q1):
    cpw = max(q0, q1)
    rpt = n_pad // NS

    @functools.partial(
        pl.kernel,
        out_type=jax.ShapeDtypeStruct((NC, n_pad, LW), jnp.float32),
        mesh=_sc_mesh(),
        scratch_types=[
            pltpu.VMEM((cpw, CHUNK), jnp.int32),
            pltpu.VMEM((CHUNK, LW), jnp.float32),
            pltpu.VMEM_SHARED((n_pad, LW), jnp.float32),
        ],
    )
    def k(dst_h, ones_h, zeros_h, out_h, dst_v, ones_v, acc):
        cid = lax.axis_index("c")
        sid = lax.axis_index("s")
        wid = sid * NC + cid
        r0 = sid * rpt
        qc = jnp.where(cid == 0, q0, q1)
        pltpu.sync_copy(dst_h.at[wid], dst_v)
        pltpu.sync_copy(ones_h, ones_v)
        pltpu.sync_copy(zeros_h.at[pl.ds(r0, rpt)], acc.at[pl.ds(r0, rpt)])
        plsc.subcore_barrier()

        def chunk(j, carry):
            pltpu.sync_copy(ones_v, acc.at[dst_v.at[j]], add=True)
            return carry

        lax.fori_loop(0, qc, chunk, 0)
        plsc.subcore_barrier()
        pltpu.sync_copy(acc.at[pl.ds(r0, rpt)], out_h.at[cid].at[pl.ds(r0, rpt)])

    return k(dst_p, ones_r, zeros_r)


def _msg_pass_sc(table, packed_p, zeros_r, n_pad, t_steps, q0, q1):
    cpw = max(q0, q1)
    rpt = n_pad // NS
    nbuf = 2

    @functools.partial(
        pl.kernel,
        out_type=jax.ShapeDtypeStruct((1, t_steps, n_pad, LW), jnp.float32),
        mesh=_sc_mesh(),
        scratch_types=[
            pltpu.VMEM((cpw, CHUNK), jnp.int32),
            pltpu.VMEM((nbuf, CHUNK), jnp.int32),
            pltpu.VMEM((nbuf, CHUNK), jnp.int32),
            pltpu.VMEM((nbuf, CHUNK, LW), jnp.float32),
            pltpu.VMEM((32, LW), jnp.float32),
            pltpu.VMEM_SHARED((n_pad, LW), jnp.float32),
            pltpu.SemaphoreType.DMA((nbuf,)),
            pltpu.SemaphoreType.DMA((nbuf,)),
        ],
    )
    def k(table_h, packed_h, zeros_h, out_h, packed_v, sbuf, dbuf, rows_v,
          zbuf, acc, gsem, ssem):
        cid = lax.axis_index("c")
        sid = lax.axis_index("s")
        wid = sid * NC + cid
        r0 = sid * rpt
        qc = jnp.where(cid == 0, q0, q1)
        pltpu.sync_copy(packed_h.at[wid], packed_v)
        pltpu.sync_copy(zeros_h.at[pl.ds(0, 32)], zbuf)
        del wid

        def unpack(j, slot):
            for kk in range(CHUNK // 16):
                pk = packed_v[j, pl.ds(kk * 16, 16)]
                sbuf[slot, pl.ds(kk * 16, 16)] = lax.shift_right_logical(pk, 16)
                dbuf[slot, pl.ds(kk * 16, 16)] = lax.bitwise_and(
                    pk, jnp.int32(0xFFFF)
                )

        def slice_body(s, carry):
            unpack(jnp.int32(0), jnp.int32(0))
            pltpu.async_copy(
                table_h.at[s].at[sbuf.at[0]], rows_v.at[0], gsem.at[0]
            )

            def zero_seg(z, carryz):
                pltpu.sync_copy(zbuf, acc.at[pl.ds(r0 + z * 32, 32)])
                return carryz

            lax.fori_loop(0, rpt // 32, zero_seg, 0)
            plsc.subcore_barrier()

            def chunk(j, carry2):
                slot = lax.rem(j, nbuf)
                nslot = lax.rem(j + 1, nbuf)

                @pl.when(j + 1 < qc)
                def _():
                    @pl.when(j >= 1)
                    def _():
                        pltpu.make_async_copy(
                            rows_v.at[nslot], acc.at[dbuf.at[nslot]],
                            ssem.at[nslot],
                        ).wait()

                    unpack(j + 1, nslot)
                    pltpu.async_copy(
                        table_h.at[s].at[sbuf.at[nslot]], rows_v.at[nslot],
                        gsem.at[nslot],
                    )

                pltpu.make_async_copy(
                    table_h.at[s].at[sbuf.at[slot]], rows_v.at[slot],
                    gsem.at[slot],
                ).wait()
                pltpu.async_copy(
                    rows_v.at[slot], acc.at[dbuf.at[slot]], ssem.at[slot],
                    add=True,
                )
                return carry2

            lax.fori_loop(0, qc, chunk, 0)
            def drain(jj, carry3):
                pltpu.make_async_copy(
                    rows_v.at[lax.rem(jj, nbuf)],
                    acc.at[dbuf.at[lax.rem(jj, nbuf)]],
                    ssem.at[lax.rem(jj, nbuf)],
                ).wait()
                return carry3

            lax.fori_loop(lax.max(qc - 2, jnp.int32(0)), qc, drain, 0)
            plsc.subcore_barrier()
            pltpu.sync_copy(
                acc.at[pl.ds(r0, rpt)], out_h.at[0].at[s].at[pl.ds(r0, rpt)]
            )
            plsc.subcore_barrier()
            return carry

        @pl.when(cid == 0)
        def _():
            lax.fori_loop(0, t_steps, slice_body, 0)

    return k(table, packed_p, zeros_r)


def _dinv_from_parts(degp):
    deg = degp[0, :, 0] + degp[1, :, 0]
    return jnp.where(deg > 0, lax.rsqrt(jnp.maximum(deg, 1e-12)), 0.0)


def _tc_pre(degp, static_pad, x_p, wst, static_b, w1a, w1b, n_pad, s_sl, h_dim):
    blk = 1024
    f_node = x_p.shape[-1]
    f_static = static_pad.shape[-1]
    t_steps = s_sl // 2

    def body(degp_r, st_r, x_r, wst_r, sb_r, w1a_r, w1b_r, out_r):
        dinv = _dinv_from_parts(degp_r[...])
        se = (
            jnp.dot(st_r[...], wst_r[...], preferred_element_type=jnp.float32)
            + sb_r[0][None, :]
        )
        base = jnp.dot(se, w1b_r[...], preferred_element_type=jnp.float32)
        x = x_r[...].reshape(s_sl * blk, f_node)
        xw = jnp.dot(x, w1a_r[...], preferred_element_type=jnp.float32)
        xw = xw.reshape(s_sl, blk, h_dim) + base[None]
        packed = jnp.concatenate([xw[:t_steps], xw[t_steps:]], axis=-1)
        out_r[...] = packed * dinv[None, :, None]

    return pl.pallas_call(
        body,
        grid=(n_pad // blk,),
        in_specs=[
            pl.BlockSpec((NC, blk, LW), lambda i: (0, i, 0)),
            pl.BlockSpec((blk, f_static), lambda i: (i, 0)),
            pl.BlockSpec((s_sl, blk, f_node), lambda i: (0, i, 0)),
            pl.BlockSpec((f_static, h_dim), lambda i: (0, 0)),
            pl.BlockSpec((1, h_dim), lambda i: (0, 0)),
            pl.BlockSpec((f_node, h_dim), lambda i: (0, 0)),
            pl.BlockSpec((h_dim, h_dim), lambda i: (0, 0)),
        ],
        out_specs=pl.BlockSpec((t_steps, blk, LW), lambda i: (0, i, 0)),
        out_shape=jax.ShapeDtypeStruct((t_steps, n_pad, LW), jnp.float32),
    )(degp, static_pad, x_p, wst, static_b, w1a, w1b)


def _tc_mid(degp, y1p, b1_2, w2_2, n_pad, t_steps, h_dim):
    blk = 512

    def body(degp_r, y_r, b1_r, w2_r, out_r):
        dinv = _dinv_from_parts(degp_r[...])
        y = y_r[0]
        h1 = jnp.maximum(y * dinv[None, :, None] + b1_r[0][None, None, :], 0.0)
        t2 = jnp.dot(
            h1.reshape(t_steps * blk, LW), w2_r[...],
            preferred_element_type=jnp.float32,
        ).reshape(t_steps, blk, LW)
        out_r[...] = t2 * dinv[None, :, None]

    return pl.pallas_call(
        body,
        grid=(n_pad // blk,),
        in_specs=[
            pl.BlockSpec((NC, blk, LW), lambda i: (0, i, 0)),
            pl.BlockSpec((1, t_steps, blk, LW), lambda i: (0, 0, i, 0)),
            pl.BlockSpec((1, LW), lambda i: (0, 0)),
            pl.BlockSpec((LW, LW), lambda i: (0, 0)),
        ],
        out_specs=pl.BlockSpec((t_steps, blk, LW), lambda i: (0, i, 0)),
        out_shape=jax.ShapeDtypeStruct((t_steps, n_pad, LW), jnp.float32),
    )(degp, y1p, b1_2, w2_2)


def _tc_lstm(degp, y2p, b2_2, wih, whh, bih, bhh, d1w, d1b, d2w, d2b,
             n_pad, batch, t_steps, h_dim, fut):
    blk = 512

    def body(degp_r, y_r, b2_r, wih_r, whh_r, bih_r, bhh_r, d1w_r, d1b_r,
             d2w_r, d2b_r, out_r):
        dinv = _dinv_from_parts(degp_r[...])
        p = y_r[...]
        y = p[0]
        xs = jnp.maximum(y * dinv[None, :, None] + b2_r[0][None, None, :], 0.0)
        bias = (bih_r[0] + bhh_r[0])[None, :]
        for b in range(batch):
            h = jnp.zeros((blk, h_dim), jnp.float32)
            c = jnp.zeros((blk, h_dim), jnp.float32)
            for t in range(t_steps):
                xt = xs[t, :, b * h_dim:(b + 1) * h_dim]
                g = (
                    jnp.dot(xt, wih_r[...], preferred_element_type=jnp.float32)
                    + jnp.dot(h, whh_r[...], preferred_element_type=jnp.float32)
                    + bias
                )
                i = jax.nn.sigmoid(g[:, 0 * h_dim:1 * h_dim])
                f = jax.nn.sigmoid(g[:, 1 * h_dim:2 * h_dim])
                gg = jnp.tanh(g[:, 2 * h_dim:3 * h_dim])
                o = jax.nn.sigmoid(g[:, 3 * h_dim:4 * h_dim])
                c = f * c + i * gg
                h = o * jnp.tanh(c)
            d = jnp.maximum(
                jnp.dot(h, d1w_r[...], preferred_element_type=jnp.float32)
                + d1b_r[0][None, :],
                0.0,
            )
            out_r[b] = (
                jnp.dot(d, d2w_r[...], preferred_element_type=jnp.float32)
                + d2b_r[0][None, :]
            )

    return pl.pallas_call(
        body,
        grid=(n_pad // blk,),
        in_specs=[
            pl.BlockSpec((NC, blk, LW), lambda i: (0, i, 0)),
            pl.BlockSpec((1, t_steps, blk, LW), lambda i: (0, 0, i, 0)),
            pl.BlockSpec((1, LW), lambda i: (0, 0)),
            pl.BlockSpec((h_dim, 4 * h_dim), lambda i: (0, 0)),
            pl.BlockSpec((h_dim, 4 * h_dim), lambda i: (0, 0)),
            pl.BlockSpec((1, 4 * h_dim), lambda i: (0, 0)),
            pl.BlockSpec((1, 4 * h_dim), lambda i: (0, 0)),
            pl.BlockSpec((h_dim, h_dim), lambda i: (0, 0)),
            pl.BlockSpec((1, h_dim), lambda i: (0, 0)),
            pl.BlockSpec((h_dim, fut), lambda i: (0, 0)),
            pl.BlockSpec((1, fut), lambda i: (0, 0)),
        ],
        out_specs=pl.BlockSpec((batch, blk, fut), lambda i: (0, i, 0)),
        out_shape=jax.ShapeDtypeStruct((batch, n_pad, fut), jnp.float32),
    )(degp, y2p, b2_2, wih, whh, bih, bhh, d1w, d1b, d2w, d2b)


def kernel(X, graph, static, static_W, static_b, W1, b1, W2, b2,
           W_ih, W_hh, b_ih, b_hh, dec1_W, dec1_b, dec2_W, dec2_b):
    batch, t_steps, n, f_node = X.shape
    e = graph.shape[1]
    h_dim = W2.shape[0]
    fut = dec2_W.shape[0]
    s_sl = batch * t_steps
    n_pad = ((n + 1 + 511) // 512) * 512

    loop = jnp.arange(n, dtype=graph.dtype)
    src = jnp.concatenate([graph[0], loop])
    dst = jnp.concatenate([graph[1], loop])
    e_tot = e + n
    tc_total = -(-e_tot // CHUNK)
    q0 = -(-tc_total // NS)
    q1 = 0
    cpw_max = max(q0, q1)
    seg = [(q0 if w % NC == 0 else q1) * CHUNK for w in range(NW)]
    pad = sum(seg) - e_tot
    srcf = jnp.concatenate([src, jnp.full((pad,), n, jnp.int32)])
    dstf = jnp.concatenate([dst, jnp.full((pad,), n, jnp.int32)])
    rows_s, rows_d = [], []
    cur = 0
    for w in range(NW):
        lw = seg[w]
        fill = cpw_max * CHUNK - lw
        rows_s.append(
            jnp.pad(srcf[cur:cur + lw], (0, fill), constant_values=n)
            .reshape(cpw_max, CHUNK))
        rows_d.append(
            jnp.pad(dstf[cur:cur + lw], (0, fill), constant_values=n)
            .reshape(cpw_max, CHUNK))
        cur += lw
    src_p = jnp.stack(rows_s)
    dst_p = jnp.stack(rows_d)
    packed_p = jnp.bitwise_or(jnp.left_shift(src_p, 16), dst_p)

    zeros_r = jnp.zeros((n_pad, LW), jnp.float32)
    ones_r = jnp.ones((CHUNK, LW), jnp.float32)

    degp = _degree_sc(dst_p, ones_r, zeros_r, n_pad, q0, q1)

    static_pad = jnp.pad(static, ((0, n_pad - n), (0, 0)))
    x_p = jnp.pad(X.reshape(s_sl, n, f_node), ((0, 0), (0, n_pad - n), (0, 0)))

    b1_2 = jnp.concatenate([b1, b1]).reshape(1, -1)
    b2_2 = jnp.concatenate([b2, b2]).reshape(1, -1)
    w2_2 = jax.scipy.linalg.block_diag(W2, W2)

    table1 = _tc_pre(
        degp, static_pad, x_p, static_W.T, static_b.reshape(1, -1),
        W1[:f_node], W1[f_node:], n_pad, s_sl, h_dim,
    )
    y1p = _msg_pass_sc(table1, packed_p, zeros_r, n_pad, t_steps, q0, q1)
    table2 = _tc_mid(degp, y1p, b1_2, w2_2, n_pad, t_steps, h_dim)
    y2p = _msg_pass_sc(table2, packed_p, zeros_r, n_pad, t_steps, q0, q1)

    out_pad = _tc_lstm(
        degp, y2p, b2_2, W_ih.T, W_hh.T,
        b_ih.reshape(1, -1), b_hh.reshape(1, -1),
        dec1_W.T, dec1_b.reshape(1, -1), dec2_W.T, dec2_b.reshape(1, -1),
        n_pad, batch, t_steps, h_dim, fut,
    )
    return jnp.transpose(out_pad[:, :n, :], (0, 2, 1))


# speedup vs baseline: 1.4132x; 1.4132x over previous
import functools

import jax
import jax.numpy as jnp
from jax import lax
from jax.experimental import pallas as pl
from jax.experimental.pallas import tpu as pltpu
from jax.experimental.pallas import tpu_sc as plsc

NC = 2
NS = 16
NW = NC * NS
CHUNK = 128
LW = 128
C0_FRAC = 0.88


def _sc_mesh():
    return plsc.VectorSubcoreMesh(core_axis_name="c", subcore_axis_name="s")


def _degree_sc(dst_p, ones_r, zeros_r, n_pad, q0, q1):
    cpw = max(q0, q1)
    rpt = n_pad // NS

    @functools.partial(
        pl.kernel,
        out_type=jax.ShapeDtypeStruct((NC, n_pad, LW), jnp.float32),
        mesh=_sc_mesh(),
        scratch_types=[
            pltpu.VMEM((cpw, CHUNK), jnp.int32),
            pltpu.VMEM((CHUNK, LW), jnp.float32),
            pltpu.VMEM_SHARED((n_pad, LW), jnp.float32),
        ],
    )
    def k(dst_h, ones_h, zeros_h, out_h, dst_v, ones_v, acc):
        cid = lax.axis_index("c")
        sid = lax.axis_index("s")
        wid = sid * NC + cid
        r0 = sid * rpt
        qc = jnp.where(cid == 0, q0, q1)
        pltpu.sync_copy(dst_h.at[wid], dst_v)
        pltpu.sync_copy(ones_h, ones_v)
        pltpu.sync_copy(zeros_h.at[pl.ds(r0, rpt)], acc.at[pl.ds(r0, rpt)])
        plsc.subcore_barrier()

        def chunk(j, carry):
            pltpu.sync_copy(ones_v, acc.at[dst_v.at[j]], add=True)
            return carry

        lax.fori_loop(0, qc, chunk, 0)
        plsc.subcore_barrier()
        pltpu.sync_copy(acc.at[pl.ds(r0, rpt)], out_h.at[cid].at[pl.ds(r0, rpt)])

    return k(dst_p, ones_r, zeros_r)


def _msg_pass_sc(table, packed_p, zeros_r, n_pad, t_steps, q0, q1):
    cpw = max(q0, q1)
    rpt = n_pad // NS
    nbuf = 2

    @functools.partial(
        pl.kernel,
        out_type=jax.ShapeDtypeStruct((NC, t_steps, n_pad, LW), jnp.float32),
        mesh=_sc_mesh(),
        scratch_types=[
            pltpu.VMEM((cpw, CHUNK), jnp.int32),
            pltpu.VMEM((nbuf, CHUNK), jnp.int32),
            pltpu.VMEM((nbuf, CHUNK), jnp.int32),
            pltpu.VMEM((nbuf, CHUNK, LW), jnp.float32),
            pltpu.VMEM((32, LW), jnp.float32),
            pltpu.VMEM_SHARED((n_pad, LW), jnp.float32),
            pltpu.SemaphoreType.DMA((nbuf,)),
            pltpu.SemaphoreType.DMA((nbuf,)),
        ],
    )
    def k(table_h, packed_h, zeros_h, out_h, packed_v, sbuf, dbuf, rows_v,
          zbuf, acc, gsem, ssem):
        cid = lax.axis_index("c")
        sid = lax.axis_index("s")
        wid = sid * NC + cid
        r0 = sid * rpt
        qc = jnp.where(cid == 0, q0, q1)
        pltpu.sync_copy(packed_h.at[wid], packed_v)
        pltpu.sync_copy(zeros_h.at[pl.ds(0, 32)], zbuf)

        def unpack(j, slot):
            for kk in range(CHUNK // 16):
                pk = packed_v[j, pl.ds(kk * 16, 16)]
                sbuf[slot, pl.ds(kk * 16, 16)] = lax.shift_right_logical(pk, 16)
                dbuf[slot, pl.ds(kk * 16, 16)] = lax.bitwise_and(
                    pk, jnp.int32(0xFFFF)
                )

        def slice_body(s, carry):
            unpack(jnp.int32(0), jnp.int32(0))
            pltpu.async_copy(
                table_h.at[s].at[sbuf.at[0]], rows_v.at[0], gsem.at[0]
            )

            def zero_seg(z, carryz):
                pltpu.sync_copy(zbuf, acc.at[pl.ds(r0 + z * 32, 32)])
                return carryz

            lax.fori_loop(0, rpt // 32, zero_seg, 0)
            plsc.subcore_barrier()

            def chunk(j, carry2):
                slot = lax.rem(j, nbuf)
                nslot = lax.rem(j + 1, nbuf)

                @pl.when(j + 1 < qc)
                def _():
                    @pl.when(j >= 1)
                    def _():
                        pltpu.make_async_copy(
                            rows_v.at[nslot], acc.at[dbuf.at[nslot]],
                            ssem.at[nslot],
                        ).wait()

                    unpack(j + 1, nslot)
                    pltpu.async_copy(
                        table_h.at[s].at[sbuf.at[nslot]], rows_v.at[nslot],
                        gsem.at[nslot],
                    )

                pltpu.make_async_copy(
                    table_h.at[s].at[sbuf.at[slot]], rows_v.at[slot],
                    gsem.at[slot],
                ).wait()
                pltpu.async_copy(
                    rows_v.at[slot], acc.at[dbuf.at[slot]], ssem.at[slot],
                    add=True,
                )
                return carry2

            lax.fori_loop(0, qc, chunk, 0)
            def drain(jj, carry3):
                pltpu.make_async_copy(
                    rows_v.at[lax.rem(jj, nbuf)],
                    acc.at[dbuf.at[lax.rem(jj, nbuf)]],
                    ssem.at[lax.rem(jj, nbuf)],
                ).wait()
                return carry3

            lax.fori_loop(lax.max(qc - 2, jnp.int32(0)), qc, drain, 0)
            plsc.subcore_barrier()
            pltpu.sync_copy(
                acc.at[pl.ds(r0, rpt)], out_h.at[cid].at[s].at[pl.ds(r0, rpt)]
            )
            plsc.subcore_barrier()
            return carry

        lax.fori_loop(0, t_steps, slice_body, 0)

    return k(table, packed_p, zeros_r)


def _dinv_from_parts(degp):
    deg = degp[0, :, 0] + degp[1, :, 0]
    return jnp.where(deg > 0, lax.rsqrt(jnp.maximum(deg, 1e-12)), 0.0)


def _tc_pre(degp, static_pad, x_p, wst, static_b, w1a, w1b, n_pad, s_sl, h_dim):
    blk = 1024
    f_node = x_p.shape[-1]
    f_static = static_pad.shape[-1]
    t_steps = s_sl // 2

    def body(degp_r, st_r, x_r, wst_r, sb_r, w1a_r, w1b_r, out_r):
        dinv = _dinv_from_parts(degp_r[...])
        se = (
            jnp.dot(st_r[...], wst_r[...], preferred_element_type=jnp.float32)
            + sb_r[0][None, :]
        )
        base = jnp.dot(se, w1b_r[...], preferred_element_type=jnp.float32)
        x = x_r[...].reshape(s_sl * blk, f_node)
        xw = jnp.dot(x, w1a_r[...], preferred_element_type=jnp.float32)
        xw = xw.reshape(s_sl, blk, h_dim) + base[None]
        packed = jnp.concatenate([xw[:t_steps], xw[t_steps:]], axis=-1)
        out_r[...] = packed * dinv[None, :, None]

    return pl.pallas_call(
        body,
        grid=(n_pad // blk,),
        in_specs=[
            pl.BlockSpec((NC, blk, LW), lambda i: (0, i, 0)),
            pl.BlockSpec((blk, f_static), lambda i: (i, 0)),
            pl.BlockSpec((s_sl, blk, f_node), lambda i: (0, i, 0)),
            pl.BlockSpec((f_static, h_dim), lambda i: (0, 0)),
            pl.BlockSpec((1, h_dim), lambda i: (0, 0)),
            pl.BlockSpec((f_node, h_dim), lambda i: (0, 0)),
            pl.BlockSpec((h_dim, h_dim), lambda i: (0, 0)),
        ],
        out_specs=pl.BlockSpec((t_steps, blk, LW), lambda i: (0, i, 0)),
        out_shape=jax.ShapeDtypeStruct((t_steps, n_pad, LW), jnp.float32),
    )(degp, static_pad, x_p, wst, static_b, w1a, w1b)


def _tc_mid(degp, y1p, b1_2, w2_2, n_pad, t_steps, h_dim):
    blk = 512

    def body(degp_r, y_r, b1_r, w2_r, out_r):
        dinv = _dinv_from_parts(degp_r[...])
        y = y_r[0] + y_r[1]
        h1 = jnp.maximum(y * dinv[None, :, None] + b1_r[0][None, None, :], 0.0)
        t2 = jnp.dot(
            h1.reshape(t_steps * blk, LW), w2_r[...],
            preferred_element_type=jnp.float32,
        ).reshape(t_steps, blk, LW)
        out_r[...] = t2 * dinv[None, :, None]

    return pl.pallas_call(
        body,
        grid=(n_pad // blk,),
        in_specs=[
            pl.BlockSpec((NC, blk, LW), lambda i: (0, i, 0)),
            pl.BlockSpec((NC, t_steps, blk, LW), lambda i: (0, 0, i, 0)),
            pl.BlockSpec((1, LW), lambda i: (0, 0)),
            pl.BlockSpec((LW, LW), lambda i: (0, 0)),
        ],
        out_specs=pl.BlockSpec((t_steps, blk, LW), lambda i: (0, i, 0)),
        out_shape=jax.ShapeDtypeStruct((t_steps, n_pad, LW), jnp.float32),
    )(degp, y1p, b1_2, w2_2)


def _tc_lstm(degp, y2p, b2_2, wih, whh, bih, bhh, d1w, d1b, d2w, d2b,
             n_pad, batch, t_steps, h_dim, fut):
    blk = 512

    def body(degp_r, y_r, b2_r, wih_r, whh_r, bih_r, bhh_r, d1w_r, d1b_r,
             d2w_r, d2b_r, out_r):
        dinv = _dinv_from_parts(degp_r[...])
        p = y_r[...]
        y = p[0] + p[1]
        xs = jnp.maximum(y * dinv[None, :, None] + b2_r[0][None, None, :], 0.0)
        bias = (bih_r[0] + bhh_r[0])[None, :]
        for b in range(batch):
            h = jnp.zeros((blk, h_dim), jnp.float32)
            c = jnp.zeros((blk, h_dim), jnp.float32)
            for t in range(t_steps):
                xt = xs[t, :, b * h_dim:(b + 1) * h_dim]
                g = (
                    jnp.dot(xt, wih_r[...], preferred_element_type=jnp.float32)
                    + jnp.dot(h, whh_r[...], preferred_element_type=jnp.float32)
                    + bias
                )
                i = jax.nn.sigmoid(g[:, 0 * h_dim:1 * h_dim])
                f = jax.nn.sigmoid(g[:, 1 * h_dim:2 * h_dim])
                gg = jnp.tanh(g[:, 2 * h_dim:3 * h_dim])
                o = jax.nn.sigmoid(g[:, 3 * h_dim:4 * h_dim])
                c = f * c + i * gg
                h = o * jnp.tanh(c)
            d = jnp.maximum(
                jnp.dot(h, d1w_r[...], preferred_element_type=jnp.float32)
                + d1b_r[0][None, :],
                0.0,
            )
            out_r[b] = (
                jnp.dot(d, d2w_r[...], preferred_element_type=jnp.float32)
                + d2b_r[0][None, :]
            )

    return pl.pallas_call(
        body,
        grid=(n_pad // blk,),
        in_specs=[
            pl.BlockSpec((NC, blk, LW), lambda i: (0, i, 0)),
            pl.BlockSpec((NC, t_steps, blk, LW), lambda i: (0, 0, i, 0)),
            pl.BlockSpec((1, LW), lambda i: (0, 0)),
            pl.BlockSpec((h_dim, 4 * h_dim), lambda i: (0, 0)),
            pl.BlockSpec((h_dim, 4 * h_dim), lambda i: (0, 0)),
            pl.BlockSpec((1, 4 * h_dim), lambda i: (0, 0)),
            pl.BlockSpec((1, 4 * h_dim), lambda i: (0, 0)),
            pl.BlockSpec((h_dim, h_dim), lambda i: (0, 0)),
            pl.BlockSpec((1, h_dim), lambda i: (0, 0)),
            pl.BlockSpec((h_dim, fut), lambda i: (0, 0)),
            pl.BlockSpec((1, fut), lambda i: (0, 0)),
        ],
        out_specs=pl.BlockSpec((batch, blk, fut), lambda i: (0, i, 0)),
        out_shape=jax.ShapeDtypeStruct((batch, n_pad, fut), jnp.float32),
    )(degp, y2p, b2_2, wih, whh, bih, bhh, d1w, d1b, d2w, d2b)


def kernel(X, graph, static, static_W, static_b, W1, b1, W2, b2,
           W_ih, W_hh, b_ih, b_hh, dec1_W, dec1_b, dec2_W, dec2_b):
    batch, t_steps, n, f_node = X.shape
    e = graph.shape[1]
    h_dim = W2.shape[0]
    fut = dec2_W.shape[0]
    s_sl = batch * t_steps
    n_pad = ((n + 1 + 511) // 512) * 512

    loop = jnp.arange(n, dtype=graph.dtype)
    src = jnp.concatenate([graph[0], loop])
    dst = jnp.concatenate([graph[1], loop])
    e_tot = e + n
    tc_total = -(-e_tot // CHUNK)
    q0 = max(1, min(tc_total // NS - 1, round(tc_total * C0_FRAC / NS)))
    q1 = max(1, -(-(tc_total - NS * q0) // NS))
    cpw_max = max(q0, q1)
    seg = [(q0 if w % NC == 0 else q1) * CHUNK for w in range(NW)]
    pad = sum(seg) - e_tot
    srcf = jnp.concatenate([src, jnp.full((pad,), n, jnp.int32)])
    dstf = jnp.concatenate([dst, jnp.full((pad,), n, jnp.int32)])
    rows_s, rows_d = [], []
    cur = 0
    for w in range(NW):
        lw = seg[w]
        fill = cpw_max * CHUNK - lw
        rows_s.append(
            jnp.pad(srcf[cur:cur + lw], (0, fill), constant_values=n)
            .reshape(cpw_max, CHUNK))
        rows_d.append(
            jnp.pad(dstf[cur:cur + lw], (0, fill), constant_values=n)
            .reshape(cpw_max, CHUNK))
        cur += lw
    src_p = jnp.stack(rows_s)
    dst_p = jnp.stack(rows_d)
    packed_p = jnp.bitwise_or(jnp.left_shift(src_p, 16), dst_p)

    zeros_r = jnp.zeros((n_pad, LW), jnp.float32)
    ones_r = jnp.ones((CHUNK, LW), jnp.float32)

    degp = _degree_sc(dst_p, ones_r, zeros_r, n_pad, q0, q1)

    static_pad = jnp.pad(static, ((0, n_pad - n), (0, 0)))
    x_p = jnp.pad(X.reshape(s_sl, n, f_node), ((0, 0), (0, n_pad - n), (0, 0)))

    b1_2 = jnp.concatenate([b1, b1]).reshape(1, -1)
    b2_2 = jnp.concatenate([b2, b2]).reshape(1, -1)
    w2_2 = jax.scipy.linalg.block_diag(W2, W2)

    table1 = _tc_pre(
        degp, static_pad, x_p, static_W.T, static_b.reshape(1, -1),
        W1[:f_node], W1[f_node:], n_pad, s_sl, h_dim,
    )
    y1p = _msg_pass_sc(table1, packed_p, zeros_r, n_pad, t_steps, q0, q1)
    table2 = _tc_mid(degp, y1p, b1_2, w2_2, n_pad, t_steps, h_dim)
    y2p = _msg_pass_sc(table2, packed_p, zeros_r, n_pad, t_steps, q0, q1)

    out_pad = _tc_lstm(
        degp, y2p, b2_2, W_ih.T, W_hh.T,
        b_ih.reshape(1, -1), b_hh.reshape(1, -1),
        dec1_W.T, dec1_b.reshape(1, -1), dec2_W.T, dec2_b.reshape(1, -1),
        n_pad, batch, t_steps, h_dim, fut,
    )
    return jnp.transpose(out_pad[:, :n, :], (0, 2, 1))
